# baseline (speedup 1.0000x reference)
import jax
import jax.numpy as jnp
from jax.experimental import pallas as pl
from jax.experimental.pallas import tpu as pltpu

_D = 768
_E = 64
_K = 8
_A = 10.0
_R = 4096


def _gating_kernel(x1_ref, x2_ref, wt_ref, b_ref, o_ref):
    x1 = x1_ref[...]
    x2 = x2_ref[...]
    wt = wt_ref[...]
    b = b_ref[...]
    logits = (jnp.dot(x1, wt[: _D // 2], preferred_element_type=jnp.float32)
              + jnp.dot(x2, wt[_D // 2 :], preferred_element_type=jnp.float32)
              + b)

    neg_inf = jnp.float32(-jnp.inf)
    cur = logits
    for _ in range(_K):
        m = jnp.max(cur, axis=1, keepdims=True)
        cur = jnp.where(cur < m, cur, neg_inf)
    mask = cur != neg_inf

    e = jnp.exp(logits)
    sm = e / jnp.sum(e, axis=1, keepdims=True)

    out = jnp.where(mask, _A * jnp.log(sm + 1.0), _A * (jnp.exp(sm) - 1.0))

    e2 = jnp.exp(out)
    o_ref[...] = e2 / jnp.sum(e2, axis=1, keepdims=True)


def kernel(x, W, b):
    n = x.shape[0]
    wt = W.T
    b2 = b.reshape(1, _E)
    return pl.pallas_call(
        _gating_kernel,
        grid=(n // _R,),
        in_specs=[
            pl.BlockSpec((_R, _D // 2), lambda i: (i, 0)),
            pl.BlockSpec((_R, _D // 2), lambda i: (i, 1)),
            pl.BlockSpec((_D, _E), lambda i: (0, 0)),
            pl.BlockSpec((1, _E), lambda i: (0, 0)),
        ],
        out_specs=pl.BlockSpec((_R, _E), lambda i: (i, 0)),
        out_shape=jax.ShapeDtypeStruct((n, _E), jnp.float32),
        compiler_params=pltpu.CompilerParams(
            dimension_semantics=("arbitrary",),
        ),
    )(x, x, wt, b2)


# speedup vs baseline: 1.0496x; 1.0496x over previous
import jax
import jax.numpy as jnp
from jax.experimental import pallas as pl
from jax.experimental.pallas import tpu as pltpu

_D = 768
_E = 64
_K = 8
_A = 10.0
_R = 4096


def _gating_kernel(x1_ref, x2_ref, wt_ref, b_ref, o_ref):
    x = jnp.concatenate([x1_ref[...], x2_ref[...]], axis=0)
    wt = wt_ref[...]
    b = b_ref[...]
    logits = jnp.dot(x, wt, preferred_element_type=jnp.float32) + b

    neg_inf = jnp.float32(-jnp.inf)
    cur = logits
    for _ in range(_K):
        m = jnp.max(cur, axis=1, keepdims=True)
        cur = jnp.where(cur < m, cur, neg_inf)
    mask = cur != neg_inf

    e = jnp.exp(logits)
    sm = e / jnp.sum(e, axis=1, keepdims=True)

    out = jnp.where(mask, _A * jnp.log(sm + 1.0), _A * (jnp.exp(sm) - 1.0))

    e2 = jnp.exp(out)
    o_ref[...] = e2 / jnp.sum(e2, axis=1, keepdims=True)


def kernel(x, W, b):
    n = x.shape[0]
    wt = W.T
    b2 = b.reshape(1, _E)
    return pl.pallas_call(
        _gating_kernel,
        grid=(n // _R,),
        in_specs=[
            pl.BlockSpec((_R // 2, _D), lambda i: (2 * i, 0)),
            pl.BlockSpec((_R // 2, _D), lambda i: (2 * i + 1, 0)),
            pl.BlockSpec((_D, _E), lambda i: (0, 0)),
            pl.BlockSpec((1, _E), lambda i: (0, 0)),
        ],
        out_specs=pl.BlockSpec((_R, _E), lambda i: (i, 0)),
        out_shape=jax.ShapeDtypeStruct((n, _E), jnp.float32),
        compiler_params=pltpu.CompilerParams(
            dimension_semantics=("arbitrary",),
        ),
    )(x, x, wt, b2)


# speedup vs baseline: 1.4677x; 1.3983x over previous
import jax
import jax.numpy as jnp
from jax.experimental import pallas as pl
from jax.experimental.pallas import tpu as pltpu

_D = 768
_E = 64
_K = 8
_A = 10.0
_R = 4096


def _gating_kernel(x_ref, w_ref, b_ref, o_ref):
    x = x_ref[...]
    w = w_ref[...]
    b = b_ref[...]
    lt = jax.lax.dot_general(
        w, x, (((1,), (1,)), ((), ())),
        preferred_element_type=jnp.float32,
    ) + b

    neg_inf = jnp.float32(-jnp.inf)
    cur = lt
    for _ in range(_K):
        m = jnp.max(cur, axis=0, keepdims=True)
        cur = jnp.where(cur < m, cur, neg_inf)
    mask = cur != neg_inf

    e = jnp.exp(lt)
    sm = e / jnp.sum(e, axis=0, keepdims=True)

    out = jnp.where(mask, _A * jnp.log(sm + 1.0), _A * (jnp.exp(sm) - 1.0))

    e2 = jnp.exp(out)
    g = e2 / jnp.sum(e2, axis=0, keepdims=True)
    o_ref[...] = g.T


def kernel(x, W, b):
    n = x.shape[0]
    b2 = b.reshape(_E, 1)
    return pl.pallas_call(
        _gating_kernel,
        grid=(n // _R,),
        in_specs=[
            pl.BlockSpec((_R, _D), lambda i: (i, 0)),
            pl.BlockSpec((_E, _D), lambda i: (0, 0)),
            pl.BlockSpec((_E, 1), lambda i: (0, 0)),
        ],
        out_specs=pl.BlockSpec((_R, _E), lambda i: (i, 0)),
        out_shape=jax.ShapeDtypeStruct((n, _E), jnp.float32),
        compiler_params=pltpu.CompilerParams(
            dimension_semantics=("arbitrary",),
        ),
    )(x, W, b2)
